# Initial kernel scaffold; baseline (speedup 1.0000x reference)
import jax
import jax.numpy as jnp
from jax.experimental import pallas as pl


def kernel(x, edge_index, W1, b1, proj_w, proj_b, prop_weight, W2, b2):
    raise NotImplementedError("write your pallas kernel here")


# speedup vs baseline: 12.1825x; 12.1825x over previous
import functools

import jax
import jax.numpy as jnp
from jax import lax
from jax.experimental import pallas as pl
from jax.experimental.pallas import tpu as pltpu
from jax.experimental.pallas import tpu_sc as plsc

N_NODES = 10000
D_FEAT = 128
HIDDEN = 64
K_HOPS = 8
NUM_CLASSES = 40

NPAD = 10240
TILE_NODES = NPAD // 16
HALF = HIDDEN // 2
CH = 512
NCHUNK = 41
EPT = CH * NCHUNK
EPAD = EPT * 16
DUMMY = N_NODES

_SCALE_CHUNKS = ((0, 160), (160, 160), (320, 160), (480, 160))


def _rsqrt(x):
    i = lax.bitcast_convert_type(x, jnp.int32)
    i = jnp.int32(0x5F3759DF) - (i >> 1)
    y = lax.bitcast_convert_type(i, jnp.float32)
    for _ in range(3):
        y = y * (1.5 - 0.5 * x * y * y)
    return y


def _sc_prop_body(h_hbm, row_hbm, col_hbm, ucur, u_out,
                  rowb, colb, gbuf, zbuf, s1, unext):
    c = lax.axis_index("c")
    sid = lax.axis_index("s")
    nb = sid * TILE_NODES

    def _zero_unext():
        for q in range(TILE_NODES // 128):
            pltpu.sync_copy(zbuf, unext.at[pl.ds(nb + q * 128, 128)])

    def _zfill(i, _):
        zbuf[i, pl.ds(0, 16)] = jnp.zeros((16,), jnp.float32)
        zbuf[i, pl.ds(16, 16)] = jnp.zeros((16,), jnp.float32)
        return 0
    lax.fori_loop(0, 128, _zfill, 0)
    _zero_unext()
    plsc.subcore_barrier()

    def _ofill(i, _):
        gbuf[i, pl.ds(0, 16)] = jnp.full((16,), 1.0, jnp.float32)
        gbuf[i, pl.ds(16, 16)] = jnp.full((16,), 1.0, jnp.float32)
        return 0
    lax.fori_loop(0, CH, _ofill, 0)

    def _dscat(j, _):
        pltpu.sync_copy(col_hbm.at[sid, j], colb)
        pltpu.sync_copy(gbuf.at[pl.ds(0, CH)], unext.at[colb], add=True)
        return 0
    lax.fori_loop(0, NCHUNK, _dscat, 0)
    plsc.subcore_barrier()

    for base, sz in _SCALE_CHUNKS:
        pltpu.sync_copy(unext.at[pl.ds(nb + base, sz)], gbuf.at[pl.ds(0, sz)])

        def _sfill(r, _, base=base):
            dv = gbuf[r, pl.ds(0, 16)]
            s1[base + r] = _rsqrt(jnp.maximum(dv[0], 1.0))
            return 0
        lax.fori_loop(0, sz, _sfill, 0)
    plsc.subcore_barrier()

    for base, sz in _SCALE_CHUNKS:
        pltpu.sync_copy(h_hbm.at[c, pl.ds(nb + base, sz)],
                        gbuf.at[pl.ds(0, sz)])

        def _ubody(r, _, base=base):
            sv = jnp.broadcast_to(s1[base + r], (16,))
            for half in (0, 16):
                gbuf[r, pl.ds(half, 16)] = gbuf[r, pl.ds(half, 16)] * sv
            return 0
        lax.fori_loop(0, sz, _ubody, 0)
        pltpu.sync_copy(gbuf.at[pl.ds(0, sz)],
                        ucur.at[c, pl.ds(nb + base, sz)])
    plsc.subcore_barrier()

    for k in range(K_HOPS):
        _zero_unext()
        plsc.subcore_barrier()

        def _ebody(j, _):
            pltpu.sync_copy(row_hbm.at[sid, j], rowb)
            pltpu.sync_copy(col_hbm.at[sid, j], colb)
            pltpu.sync_copy(ucur.at[c].at[rowb], gbuf)
            pltpu.sync_copy(gbuf, unext.at[colb], add=True)
            return 0
        lax.fori_loop(0, NCHUNK, _ebody, 0)
        plsc.subcore_barrier()

        for base, sz in _SCALE_CHUNKS:
            pltpu.sync_copy(unext.at[pl.ds(nb + base, sz)],
                            gbuf.at[pl.ds(0, sz)])

            def _sbody(r, _, base=base):
                sv = jnp.broadcast_to(s1[base + r], (16,))
                for half in (0, 16):
                    av = gbuf[r, pl.ds(half, 16)]
                    cv = av * sv
                    gbuf[256 + r, pl.ds(half, 16)] = cv
                    gbuf[r, pl.ds(half, 16)] = cv * sv
                return 0
            lax.fori_loop(0, sz, _sbody, 0)
            pltpu.sync_copy(gbuf.at[pl.ds(0, sz)],
                            ucur.at[c, pl.ds(nb + base, sz)])
            pltpu.sync_copy(gbuf.at[pl.ds(256, sz)],
                            u_out.at[k, c, pl.ds(nb + base, sz)])
        plsc.subcore_barrier()


def _sc_prop(h_split, row, col):
    mesh = plsc.VectorSubcoreMesh(core_axis_name="c", subcore_axis_name="s")
    fn = functools.partial(
        pl.kernel,
        mesh=mesh,
        compiler_params=pltpu.CompilerParams(use_tc_tiling_on_sc=False),
        out_type=jax.ShapeDtypeStruct((K_HOPS, 2, NPAD, HALF), jnp.float32),
        scratch_types=[
            pltpu.VMEM((CH,), jnp.int32),
            pltpu.VMEM((CH,), jnp.int32),
            pltpu.VMEM((CH, HALF), jnp.float32),
            pltpu.VMEM((128, HALF), jnp.float32),
            pltpu.SMEM((TILE_NODES,), jnp.float32),
            pltpu.VMEM_SHARED((NPAD, HALF), jnp.float32),
        ],
    )(_sc_prop_body)
    ucur_ref = jax.new_ref(jnp.zeros((2, NPAD, HALF), jnp.float32))
    return fn(h_split, row, col, ucur_ref)


_BN = 1280


def _pre_body(x_ref, w_ref, b_ref, h_ref):
    h = jnp.dot(x_ref[...], w_ref[...],
                preferred_element_type=jnp.float32) + b_ref[...]
    alpha = 1.6732632423543772848170429916717
    scale = 1.0507009873554804934193349852946
    neg = alpha * (jnp.exp(jnp.minimum(h, 0.0)) - 1.0)
    h_ref[...] = scale * jnp.where(h > 0.0, h, neg)


def _tc_pre(x_pad, W1, b1):
    return pl.pallas_call(
        _pre_body,
        grid=(NPAD // _BN,),
        in_specs=[
            pl.BlockSpec((_BN, D_FEAT), lambda i: (i, 0)),
            pl.BlockSpec((D_FEAT, HIDDEN), lambda i: (0, 0)),
            pl.BlockSpec((1, HIDDEN), lambda i: (0, 0)),
        ],
        out_specs=pl.BlockSpec((_BN, HIDDEN), lambda i: (i, 0)),
        out_shape=jax.ShapeDtypeStruct((NPAD, HIDDEN), jnp.float32),
    )(x_pad, W1, b1.reshape(1, HIDDEN))


def _postA_body(pps_ref, pw_ref, pb_ref, fp_ref, cs_ref):
    i = pl.program_id(0)
    pps = pps_ref[...]
    pw = pw_ref[...]
    fp = jnp.tensordot(pps, pw[:, 0], axes=[[2], [0]])
    fp = fp.T + pb_ref[0, 0]
    rows = i * _BN + lax.broadcasted_iota(jnp.int32, (_BN, 1), 0)
    valid = rows < N_NODES
    part = jnp.sum(jnp.where(valid, fp, 0.0), axis=0, keepdims=True)

    @pl.when(i == 0)
    def _():
        cs_ref[...] = jnp.zeros_like(cs_ref)
    cs_ref[...] += part
    fp_ref[...] = fp


def _tc_postA(pps, proj_w, proj_b):
    nk = K_HOPS + 1
    return pl.pallas_call(
        _postA_body,
        grid=(NPAD // _BN,),
        in_specs=[
            pl.BlockSpec((nk, _BN, HIDDEN), lambda i: (0, i, 0)),
            pl.BlockSpec((HIDDEN, 1), lambda i: (0, 0)),
            pl.BlockSpec((1, 1), lambda i: (0, 0)),
        ],
        out_specs=[
            pl.BlockSpec((_BN, nk), lambda i: (i, 0)),
            pl.BlockSpec((1, nk), lambda i: (0, 0)),
        ],
        out_shape=[
            jax.ShapeDtypeStruct((NPAD, nk), jnp.float32),
            jax.ShapeDtypeStruct((1, nk), jnp.float32),
        ],
    )(pps, proj_w, proj_b.reshape(1, 1))


def _postB_body(pps_ref, fp_ref, cs_ref, prop_ref, w2_ref, b2_ref, out_ref):
    nk = K_HOPS + 1
    fp = fp_ref[...]
    m = cs_ref[...] / N_NODES
    sw = m + prop_ref[...]
    sw = sw / jnp.maximum(jnp.sqrt(jnp.sum(sw * sw)), 1e-12)
    sn = jnp.sqrt(jnp.sum(fp * fp, axis=1, keepdims=True))
    selfw = fp / jnp.maximum(sn, 1e-12)
    share = jnp.zeros((_BN, HIDDEN), jnp.float32)
    selfr = jnp.zeros((_BN, HIDDEN), jnp.float32)
    for k in range(nk):
        pk = pps_ref[k]
        share = share + sw[0, k] * pk
        selfr = selfr + selfw[:, k:k + 1] * pk
    out = share * 0.2 + selfr * 0.8
    t = jnp.tanh(out)
    logits = jnp.dot(t, w2_ref[...],
                     preferred_element_type=jnp.float32) + b2_ref[...]
    lmax = jnp.max(logits, axis=1, keepdims=True)
    ex = jnp.exp(logits - lmax)
    out_ref[...] = logits - lmax - jnp.log(jnp.sum(ex, axis=1, keepdims=True))


def _tc_postB(pps, fp, cs, prop_weight, W2, b2):
    nk = K_HOPS + 1
    return pl.pallas_call(
        _postB_body,
        grid=(NPAD // _BN,),
        in_specs=[
            pl.BlockSpec((nk, _BN, HIDDEN), lambda i: (0, i, 0)),
            pl.BlockSpec((_BN, nk), lambda i: (i, 0)),
            pl.BlockSpec((1, nk), lambda i: (0, 0)),
            pl.BlockSpec((1, nk), lambda i: (0, 0)),
            pl.BlockSpec((HIDDEN, NUM_CLASSES), lambda i: (0, 0)),
            pl.BlockSpec((1, NUM_CLASSES), lambda i: (0, 0)),
        ],
        out_specs=pl.BlockSpec((_BN, NUM_CLASSES), lambda i: (i, 0)),
        out_shape=jax.ShapeDtypeStruct((NPAD, NUM_CLASSES), jnp.float32),
    )(pps, fp, cs, prop_weight, W2, b2.reshape(1, NUM_CLASSES))


def kernel(x, edge_index, W1, b1, proj_w, proj_b, prop_weight, W2, b2):
    ei = edge_index.astype(jnp.int32)
    n_edges = ei.shape[1]
    n_real = n_edges + N_NODES
    loop = jnp.arange(N_NODES, dtype=jnp.int32)
    row = jnp.concatenate(
        [ei[0], loop, jnp.zeros((EPAD - n_real,), jnp.int32)])
    col = jnp.concatenate(
        [ei[1], loop, jnp.full((EPAD - n_real,), DUMMY, jnp.int32)])
    row = row.reshape(16, NCHUNK, CH)
    col = col.reshape(16, NCHUNK, CH)

    x_pad = jnp.pad(x, ((0, NPAD - N_NODES), (0, 0)))
    h_pad = _tc_pre(x_pad, W1, b1)
    h_split = h_pad.reshape(NPAD, 2, HALF).transpose(1, 0, 2)

    u4 = _sc_prop(h_split, row, col)
    u_full = u4.transpose(0, 2, 1, 3).reshape(K_HOPS, NPAD, HIDDEN)
    pps = jnp.concatenate([h_pad[None], u_full], axis=0)

    fp, cs = _tc_postA(pps, proj_w, proj_b)
    out = _tc_postB(pps, fp, cs, prop_weight, W2, b2)
    return out[:N_NODES]


# speedup vs baseline: 18.0606x; 1.4825x over previous
import functools

import jax
import jax.numpy as jnp
from jax import lax
from jax.experimental import pallas as pl
from jax.experimental.pallas import tpu as pltpu
from jax.experimental.pallas import tpu_sc as plsc

N_NODES = 10000
D_FEAT = 128
HIDDEN = 64
K_HOPS = 8
NUM_CLASSES = 40

NPAD = 10240
TILE_NODES = NPAD // 16
HALF = HIDDEN // 2
CH = 512
NCHUNK = 41
EPT = CH * NCHUNK
EPAD = EPT * 16
DUMMY = N_NODES

_SCALE_CHUNKS = ((0, 160), (160, 160), (320, 160), (480, 160))


def _rsqrt(x):
    i = lax.bitcast_convert_type(x, jnp.int32)
    i = jnp.int32(0x5F3759DF) - (i >> 1)
    y = lax.bitcast_convert_type(i, jnp.float32)
    for _ in range(3):
        y = y * (1.5 - 0.5 * x * y * y)
    return y


def _sc_prop_body(h_hbm, row_hbm, col_hbm, ucur, u_out,
                  rowv, colv, gbuf, zbuf, s1, unext, gsem, ssem):
    c = lax.axis_index("c")
    sid = lax.axis_index("s")
    nb = sid * TILE_NODES

    def _wait_g(p):
        pltpu.make_async_copy(ucur.at[c, pl.ds(0, CH)], gbuf.at[p],
                              gsem.at[p]).wait()

    def _wait_s(p, sz=CH):
        pltpu.make_async_copy(gbuf.at[p, pl.ds(0, sz)],
                              unext.at[pl.ds(0, sz)], ssem.at[p]).wait()

    pltpu.sync_copy(row_hbm.at[sid], rowv)
    pltpu.sync_copy(col_hbm.at[sid], colv)

    def _zfill(i, _):
        zbuf[i, pl.ds(0, 16)] = jnp.zeros((16,), jnp.float32)
        zbuf[i, pl.ds(16, 16)] = jnp.zeros((16,), jnp.float32)
        return 0
    lax.fori_loop(0, TILE_NODES, _zfill, 0)
    pltpu.sync_copy(zbuf, unext.at[pl.ds(nb, TILE_NODES)])
    plsc.subcore_barrier()

    def _ofill(i, _):
        gbuf[0, i, pl.ds(0, 16)] = jnp.full((16,), 1.0, jnp.float32)
        gbuf[0, i, pl.ds(16, 16)] = jnp.full((16,), 1.0, jnp.float32)
        return 0
    lax.fori_loop(0, CH, _ofill, 0)

    def _dscat(j, _):
        p = j % 2

        @pl.when(j >= 2)
        def _():
            _wait_s(p)
        pltpu.async_copy(gbuf.at[0], unext.at[colv.at[j]], ssem.at[p],
                         add=True)
        return 0
    lax.fori_loop(0, NCHUNK, _dscat, 0)
    _wait_s((NCHUNK - 2) % 2)
    _wait_s((NCHUNK - 1) % 2)
    plsc.subcore_barrier()

    pltpu.async_copy(unext.at[pl.ds(nb, 160)], gbuf.at[0, pl.ds(0, 160)],
                     gsem.at[0])
    for q, (base, sz) in enumerate(_SCALE_CHUNKS):
        p = q % 2
        pltpu.make_async_copy(unext.at[pl.ds(0, sz)],
                              gbuf.at[p, pl.ds(0, sz)], gsem.at[p]).wait()
        if q + 1 < len(_SCALE_CHUNKS):
            nbase, nsz = _SCALE_CHUNKS[q + 1]
            pltpu.async_copy(unext.at[pl.ds(nb + nbase, nsz)],
                             gbuf.at[1 - p, pl.ds(0, nsz)], gsem.at[1 - p])

        def _sfill(r, _, base=base, p=p):
            dv = gbuf[p, r, pl.ds(0, 16)]
            s1[base + r] = _rsqrt(jnp.maximum(dv[0], 1.0))
            return 0
        lax.fori_loop(0, sz, _sfill, 0)
    plsc.subcore_barrier()

    pltpu.async_copy(h_hbm.at[c, pl.ds(nb, 160)], gbuf.at[0, pl.ds(0, 160)],
                     gsem.at[0])
    for q, (base, sz) in enumerate(_SCALE_CHUNKS):
        p = q % 2
        pltpu.make_async_copy(h_hbm.at[c, pl.ds(0, sz)],
                              gbuf.at[p, pl.ds(0, sz)], gsem.at[p]).wait()
        if q >= 1:
            _wait_s(1 - p, 160)
        if q + 1 < len(_SCALE_CHUNKS):
            nbase, nsz = _SCALE_CHUNKS[q + 1]
            pltpu.async_copy(h_hbm.at[c, pl.ds(nb + nbase, nsz)],
                             gbuf.at[1 - p, pl.ds(0, nsz)], gsem.at[1 - p])

        def _ubody(r, _, base=base, p=p):
            sv = jnp.broadcast_to(s1[base + r], (16,))
            for half in (0, 16):
                gbuf[p, r, pl.ds(half, 16)] = gbuf[p, r, pl.ds(half, 16)] * sv
            return 0
        lax.fori_loop(0, sz, _ubody, 0)
        pltpu.async_copy(gbuf.at[p, pl.ds(0, sz)],
                         ucur.at[c, pl.ds(nb + base, sz)], ssem.at[p])
    _wait_s(1, 160)
    plsc.subcore_barrier()

    for k in range(K_HOPS):
        pltpu.sync_copy(zbuf, unext.at[pl.ds(nb, TILE_NODES)])
        plsc.subcore_barrier()

        pltpu.async_copy(ucur.at[c].at[rowv.at[0]], gbuf.at[0], gsem.at[0])

        def _ebody(j, _):
            p = j % 2
            pn = (j + 1) % 2

            @pl.when(j + 1 < NCHUNK)
            def _():
                @pl.when(j >= 1)
                def _():
                    _wait_s(pn)
                pltpu.async_copy(ucur.at[c].at[rowv.at[j + 1]], gbuf.at[pn],
                                 gsem.at[pn])
            _wait_g(p)
            pltpu.async_copy(gbuf.at[p], unext.at[colv.at[j]], ssem.at[p],
                             add=True)
            return 0
        lax.fori_loop(0, NCHUNK, _ebody, 0)
        _wait_s((NCHUNK - 2) % 2)
        _wait_s((NCHUNK - 1) % 2)
        plsc.subcore_barrier()

        pltpu.async_copy(unext.at[pl.ds(nb, 160)], gbuf.at[0, pl.ds(0, 160)],
                         gsem.at[0])
        for q, (base, sz) in enumerate(_SCALE_CHUNKS):
            p = q % 2
            pltpu.make_async_copy(unext.at[pl.ds(0, sz)],
                                  gbuf.at[p, pl.ds(0, sz)], gsem.at[p]).wait()
            if q >= 1:
                _wait_s(1 - p, 160)
                _wait_s(1 - p, 160)
            if q + 1 < len(_SCALE_CHUNKS):
                nbase, nsz = _SCALE_CHUNKS[q + 1]
                pltpu.async_copy(unext.at[pl.ds(nb + nbase, nsz)],
                                 gbuf.at[1 - p, pl.ds(0, nsz)],
                                 gsem.at[1 - p])

            def _sbody(r, _, base=base, p=p):
                sv = jnp.broadcast_to(s1[base + r], (16,))
                for half in (0, 16):
                    av = gbuf[p, r, pl.ds(half, 16)]
                    cv = av * sv
                    gbuf[p, 256 + r, pl.ds(half, 16)] = cv
                    gbuf[p, r, pl.ds(half, 16)] = cv * sv
                return 0
            lax.fori_loop(0, sz, _sbody, 0)
            pltpu.async_copy(gbuf.at[p, pl.ds(0, sz)],
                             ucur.at[c, pl.ds(nb + base, sz)], ssem.at[p])
            pltpu.async_copy(gbuf.at[p, pl.ds(256, sz)],
                             u_out.at[k, c, pl.ds(nb + base, sz)], ssem.at[p])
        _wait_s(1, 160)
        _wait_s(1, 160)
        plsc.subcore_barrier()


def _sc_prop(h_split, row, col):
    mesh = plsc.VectorSubcoreMesh(core_axis_name="c", subcore_axis_name="s")
    fn = functools.partial(
        pl.kernel,
        mesh=mesh,
        compiler_params=pltpu.CompilerParams(use_tc_tiling_on_sc=False),
        out_type=jax.ShapeDtypeStruct((K_HOPS, 2, NPAD, HALF), jnp.float32),
        scratch_types=[
            pltpu.VMEM((NCHUNK, CH), jnp.int32),
            pltpu.VMEM((NCHUNK, CH), jnp.int32),
            pltpu.VMEM((2, CH, HALF), jnp.float32),
            pltpu.VMEM((TILE_NODES, HALF), jnp.float32),
            pltpu.SMEM((TILE_NODES,), jnp.float32),
            pltpu.VMEM_SHARED((NPAD, HALF), jnp.float32),
            pltpu.SemaphoreType.DMA((2,)),
            pltpu.SemaphoreType.DMA((2,)),
        ],
    )(_sc_prop_body)
    ucur_ref = jax.new_ref(jnp.zeros((2, NPAD, HALF), jnp.float32))
    return fn(h_split, row, col, ucur_ref)


_BN = 1280


def _pre_body(x_ref, w_ref, b_ref, h_ref):
    h = jnp.dot(x_ref[...], w_ref[...],
                preferred_element_type=jnp.float32) + b_ref[...]
    alpha = 1.6732632423543772848170429916717
    scale = 1.0507009873554804934193349852946
    neg = alpha * (jnp.exp(jnp.minimum(h, 0.0)) - 1.0)
    h_ref[...] = scale * jnp.where(h > 0.0, h, neg)


def _tc_pre(x_pad, W1, b1):
    return pl.pallas_call(
        _pre_body,
        grid=(NPAD // _BN,),
        in_specs=[
            pl.BlockSpec((_BN, D_FEAT), lambda i: (i, 0)),
            pl.BlockSpec((D_FEAT, HIDDEN), lambda i: (0, 0)),
            pl.BlockSpec((1, HIDDEN), lambda i: (0, 0)),
        ],
        out_specs=pl.BlockSpec((_BN, HIDDEN), lambda i: (i, 0)),
        out_shape=jax.ShapeDtypeStruct((NPAD, HIDDEN), jnp.float32),
    )(x_pad, W1, b1.reshape(1, HIDDEN))


def _postA_body(pps_ref, pw_ref, pb_ref, fp_ref, cs_ref):
    i = pl.program_id(0)
    pps = pps_ref[...]
    pw = pw_ref[...]
    fp = jnp.tensordot(pps, pw[:, 0], axes=[[2], [0]])
    fp = fp.T + pb_ref[0, 0]
    rows = i * _BN + lax.broadcasted_iota(jnp.int32, (_BN, 1), 0)
    valid = rows < N_NODES
    part = jnp.sum(jnp.where(valid, fp, 0.0), axis=0, keepdims=True)

    @pl.when(i == 0)
    def _():
        cs_ref[...] = jnp.zeros_like(cs_ref)
    cs_ref[...] += part
    fp_ref[...] = fp


def _tc_postA(pps, proj_w, proj_b):
    nk = K_HOPS + 1
    return pl.pallas_call(
        _postA_body,
        grid=(NPAD // _BN,),
        in_specs=[
            pl.BlockSpec((nk, _BN, HIDDEN), lambda i: (0, i, 0)),
            pl.BlockSpec((HIDDEN, 1), lambda i: (0, 0)),
            pl.BlockSpec((1, 1), lambda i: (0, 0)),
        ],
        out_specs=[
            pl.BlockSpec((_BN, nk), lambda i: (i, 0)),
            pl.BlockSpec((1, nk), lambda i: (0, 0)),
        ],
        out_shape=[
            jax.ShapeDtypeStruct((NPAD, nk), jnp.float32),
            jax.ShapeDtypeStruct((1, nk), jnp.float32),
        ],
    )(pps, proj_w, proj_b.reshape(1, 1))


def _postB_body(pps_ref, fp_ref, cs_ref, prop_ref, w2_ref, b2_ref, out_ref):
    nk = K_HOPS + 1
    fp = fp_ref[...]
    m = cs_ref[...] / N_NODES
    sw = m + prop_ref[...]
    sw = sw / jnp.maximum(jnp.sqrt(jnp.sum(sw * sw)), 1e-12)
    sn = jnp.sqrt(jnp.sum(fp * fp, axis=1, keepdims=True))
    selfw = fp / jnp.maximum(sn, 1e-12)
    share = jnp.zeros((_BN, HIDDEN), jnp.float32)
    selfr = jnp.zeros((_BN, HIDDEN), jnp.float32)
    for k in range(nk):
        pk = pps_ref[k]
        share = share + sw[0, k] * pk
        selfr = selfr + selfw[:, k:k + 1] * pk
    out = share * 0.2 + selfr * 0.8
    t = jnp.tanh(out)
    logits = jnp.dot(t, w2_ref[...],
                     preferred_element_type=jnp.float32) + b2_ref[...]
    lmax = jnp.max(logits, axis=1, keepdims=True)
    ex = jnp.exp(logits - lmax)
    out_ref[...] = logits - lmax - jnp.log(jnp.sum(ex, axis=1, keepdims=True))


def _tc_postB(pps, fp, cs, prop_weight, W2, b2):
    nk = K_HOPS + 1
    return pl.pallas_call(
        _postB_body,
        grid=(NPAD // _BN,),
        in_specs=[
            pl.BlockSpec((nk, _BN, HIDDEN), lambda i: (0, i, 0)),
            pl.BlockSpec((_BN, nk), lambda i: (i, 0)),
            pl.BlockSpec((1, nk), lambda i: (0, 0)),
            pl.BlockSpec((1, nk), lambda i: (0, 0)),
            pl.BlockSpec((HIDDEN, NUM_CLASSES), lambda i: (0, 0)),
            pl.BlockSpec((1, NUM_CLASSES), lambda i: (0, 0)),
        ],
        out_specs=pl.BlockSpec((_BN, NUM_CLASSES), lambda i: (i, 0)),
        out_shape=jax.ShapeDtypeStruct((NPAD, NUM_CLASSES), jnp.float32),
    )(pps, fp, cs, prop_weight, W2, b2.reshape(1, NUM_CLASSES))


def kernel(x, edge_index, W1, b1, proj_w, proj_b, prop_weight, W2, b2):
    ei = edge_index.astype(jnp.int32)
    n_edges = ei.shape[1]
    n_real = n_edges + N_NODES
    loop = jnp.arange(N_NODES, dtype=jnp.int32)
    row = jnp.concatenate(
        [ei[0], loop, jnp.zeros((EPAD - n_real,), jnp.int32)])
    col = jnp.concatenate(
        [ei[1], loop, jnp.full((EPAD - n_real,), DUMMY, jnp.int32)])
    row = row.reshape(16, NCHUNK, CH)
    col = col.reshape(16, NCHUNK, CH)

    x_pad = jnp.pad(x, ((0, NPAD - N_NODES), (0, 0)))
    h_pad = _tc_pre(x_pad, W1, b1)
    h_split = h_pad.reshape(NPAD, 2, HALF).transpose(1, 0, 2)

    u4 = _sc_prop(h_split, row, col)
    u_full = u4.transpose(0, 2, 1, 3).reshape(K_HOPS, NPAD, HIDDEN)
    pps = jnp.concatenate([h_pad[None], u_full], axis=0)

    fp, cs = _tc_postA(pps, proj_w, proj_b)
    out = _tc_postB(pps, fp, cs, prop_weight, W2, b2)
    return out[:N_NODES]


# speedup vs baseline: 18.6074x; 1.0303x over previous
import functools

import jax
import jax.numpy as jnp
from jax import lax
from jax.experimental import pallas as pl
from jax.experimental.pallas import tpu as pltpu
from jax.experimental.pallas import tpu_sc as plsc

N_NODES = 10000
D_FEAT = 128
HIDDEN = 64
K_HOPS = 8
NUM_CLASSES = 40

NPAD = 10240
TILE_NODES = NPAD // 16
HALF = HIDDEN // 2
CH = 512
NCHUNK = 41
EPT = CH * NCHUNK
EPAD = EPT * 16
DUMMY = N_NODES

_SCALE_CHUNKS = ((0, 160), (160, 160), (320, 160), (480, 160))


def _rsqrt(x):
    i = lax.bitcast_convert_type(x, jnp.int32)
    i = jnp.int32(0x5F3759DF) - (i >> 1)
    y = lax.bitcast_convert_type(i, jnp.float32)
    for _ in range(3):
        y = y * (1.5 - 0.5 * x * y * y)
    return y


def _sc_prop_body(h_hbm, row_hbm, col_hbm, ucur, u_out,
                  rowv, colv, gbuf, zbuf, s1, unext, gsem, ssem):
    c = lax.axis_index("c")
    sid = lax.axis_index("s")
    nb = sid * TILE_NODES

    def _wait_g(p):
        pltpu.make_async_copy(ucur.at[c, pl.ds(0, CH)], gbuf.at[p],
                              gsem.at[p]).wait()

    def _wait_s(p, sz=CH):
        pltpu.make_async_copy(gbuf.at[p, pl.ds(0, sz)],
                              unext.at[pl.ds(0, sz)], ssem.at[p]).wait()

    pltpu.sync_copy(row_hbm.at[sid], rowv)
    pltpu.sync_copy(col_hbm.at[sid], colv)

    def _zfill(i, _):
        zbuf[i, pl.ds(0, 16)] = jnp.zeros((16,), jnp.float32)
        zbuf[i, pl.ds(16, 16)] = jnp.zeros((16,), jnp.float32)
        return 0
    lax.fori_loop(0, 128, _zfill, 0)

    def _zero_unext():
        for q in range(TILE_NODES // 128):
            pltpu.async_copy(zbuf, unext.at[pl.ds(nb + q * 128, 128)],
                             gsem.at[2])
        for q in range(TILE_NODES // 128):
            pltpu.make_async_copy(zbuf, unext.at[pl.ds(nb, 128)],
                                  gsem.at[2]).wait()

    _zero_unext()
    plsc.subcore_barrier()

    def _ofill(i, _):
        gbuf[0, i, pl.ds(0, 16)] = jnp.full((16,), 1.0, jnp.float32)
        gbuf[0, i, pl.ds(16, 16)] = jnp.full((16,), 1.0, jnp.float32)
        return 0
    lax.fori_loop(0, CH, _ofill, 0)

    def _dscat(j, _):
        p = j % 2

        @pl.when(j >= 2)
        def _():
            _wait_s(p)
        pltpu.async_copy(gbuf.at[0], unext.at[colv.at[j]], ssem.at[p],
                         add=True)
        return 0
    lax.fori_loop(0, NCHUNK, _dscat, 0)
    _wait_s((NCHUNK - 2) % 2)
    _wait_s((NCHUNK - 1) % 2)
    plsc.subcore_barrier()

    pltpu.async_copy(unext.at[pl.ds(nb, 160)], gbuf.at[0, pl.ds(0, 160)],
                     gsem.at[0])
    for q, (base, sz) in enumerate(_SCALE_CHUNKS):
        p = q % 2
        pltpu.make_async_copy(unext.at[pl.ds(0, sz)],
                              gbuf.at[p, pl.ds(0, sz)], gsem.at[p]).wait()
        if q + 1 < len(_SCALE_CHUNKS):
            nbase, nsz = _SCALE_CHUNKS[q + 1]
            pltpu.async_copy(unext.at[pl.ds(nb + nbase, nsz)],
                             gbuf.at[1 - p, pl.ds(0, nsz)], gsem.at[1 - p])

        def _sfill(r, _, base=base, p=p):
            dv = gbuf[p, r, pl.ds(0, 16)]
            s1[base + r] = _rsqrt(jnp.maximum(dv[0], 1.0))
            return 0
        lax.fori_loop(0, sz, _sfill, 0)
    plsc.subcore_barrier()

    pltpu.async_copy(h_hbm.at[c, pl.ds(nb, 160)], gbuf.at[0, pl.ds(0, 160)],
                     gsem.at[0])
    for q, (base, sz) in enumerate(_SCALE_CHUNKS):
        p = q % 2
        pltpu.make_async_copy(h_hbm.at[c, pl.ds(0, sz)],
                              gbuf.at[p, pl.ds(0, sz)], gsem.at[p]).wait()
        if q >= 1:
            _wait_s(1 - p, 160)
        if q + 1 < len(_SCALE_CHUNKS):
            nbase, nsz = _SCALE_CHUNKS[q + 1]
            pltpu.async_copy(h_hbm.at[c, pl.ds(nb + nbase, nsz)],
                             gbuf.at[1 - p, pl.ds(0, nsz)], gsem.at[1 - p])

        def _ubody(r, _, base=base, p=p):
            sv = jnp.broadcast_to(s1[base + r], (16,))
            for half in (0, 16):
                gbuf[p, r, pl.ds(half, 16)] = gbuf[p, r, pl.ds(half, 16)] * sv
            return 0
        lax.fori_loop(0, sz, _ubody, 0)
        pltpu.async_copy(gbuf.at[p, pl.ds(0, sz)],
                         ucur.at[c, pl.ds(nb + base, sz)], ssem.at[p])
    _wait_s(1, 160)
    plsc.subcore_barrier()

    for k in range(K_HOPS):
        _zero_unext()
        plsc.subcore_barrier()

        pltpu.async_copy(ucur.at[c].at[rowv.at[0]], gbuf.at[0], gsem.at[0])
        pltpu.async_copy(ucur.at[c].at[rowv.at[1]], gbuf.at[1], gsem.at[1])

        def _ebody(j, _):
            p = j % 3
            pn = (j + 2) % 3

            @pl.when(j + 2 < NCHUNK)
            def _():
                @pl.when(j >= 1)
                def _():
                    _wait_s(pn)
                pltpu.async_copy(ucur.at[c].at[rowv.at[j + 2]], gbuf.at[pn],
                                 gsem.at[pn])
            _wait_g(p)
            pltpu.async_copy(gbuf.at[p], unext.at[colv.at[j]], ssem.at[p],
                             add=True)
            return 0
        lax.fori_loop(0, NCHUNK, _ebody, 0)
        for jj in (NCHUNK - 3, NCHUNK - 2, NCHUNK - 1):
            _wait_s(jj % 3)
        plsc.subcore_barrier()

        pltpu.async_copy(unext.at[pl.ds(nb, 160)], gbuf.at[0, pl.ds(0, 160)],
                         gsem.at[0])
        for q, (base, sz) in enumerate(_SCALE_CHUNKS):
            p = q % 2
            pltpu.make_async_copy(unext.at[pl.ds(0, sz)],
                                  gbuf.at[p, pl.ds(0, sz)], gsem.at[p]).wait()
            if q >= 1:
                _wait_s(1 - p, 160)
                _wait_s(1 - p, 160)
            if q + 1 < len(_SCALE_CHUNKS):
                nbase, nsz = _SCALE_CHUNKS[q + 1]
                pltpu.async_copy(unext.at[pl.ds(nb + nbase, nsz)],
                                 gbuf.at[1 - p, pl.ds(0, nsz)],
                                 gsem.at[1 - p])

            def _sbody(r, _, base=base, p=p):
                sv = jnp.broadcast_to(s1[base + r], (16,))
                for half in (0, 16):
                    av = gbuf[p, r, pl.ds(half, 16)]
                    cv = av * sv
                    gbuf[p, 256 + r, pl.ds(half, 16)] = cv
                    gbuf[p, r, pl.ds(half, 16)] = cv * sv
                return 0
            lax.fori_loop(0, sz, _sbody, 0)
            pltpu.async_copy(gbuf.at[p, pl.ds(0, sz)],
                             ucur.at[c, pl.ds(nb + base, sz)], ssem.at[p])
            pltpu.async_copy(gbuf.at[p, pl.ds(256, sz)],
                             u_out.at[k, c, pl.ds(nb + base, sz)], ssem.at[p])
        _wait_s(1, 160)
        _wait_s(1, 160)
        plsc.subcore_barrier()


def _sc_prop(h_split, row, col):
    mesh = plsc.VectorSubcoreMesh(core_axis_name="c", subcore_axis_name="s")
    fn = functools.partial(
        pl.kernel,
        mesh=mesh,
        compiler_params=pltpu.CompilerParams(use_tc_tiling_on_sc=False),
        out_type=jax.ShapeDtypeStruct((K_HOPS, 2, NPAD, HALF), jnp.float32),
        scratch_types=[
            pltpu.VMEM((NCHUNK, CH), jnp.int32),
            pltpu.VMEM((NCHUNK, CH), jnp.int32),
            pltpu.VMEM((3, CH, HALF), jnp.float32),
            pltpu.VMEM((128, HALF), jnp.float32),
            pltpu.SMEM((TILE_NODES,), jnp.float32),
            pltpu.VMEM_SHARED((NPAD, HALF), jnp.float32),
            pltpu.SemaphoreType.DMA((3,)),
            pltpu.SemaphoreType.DMA((3,)),
        ],
    )(_sc_prop_body)
    ucur_ref = jax.new_ref(jnp.zeros((2, NPAD, HALF), jnp.float32))
    return fn(h_split, row, col, ucur_ref)


_BN = 1280


def _pre_body(x_ref, w_ref, b_ref, h_ref):
    h = jnp.dot(x_ref[...], w_ref[...],
                preferred_element_type=jnp.float32) + b_ref[...]
    alpha = 1.6732632423543772848170429916717
    scale = 1.0507009873554804934193349852946
    neg = alpha * (jnp.exp(jnp.minimum(h, 0.0)) - 1.0)
    h_ref[...] = scale * jnp.where(h > 0.0, h, neg)


def _tc_pre(x_pad, W1, b1):
    return pl.pallas_call(
        _pre_body,
        grid=(NPAD // _BN,),
        in_specs=[
            pl.BlockSpec((_BN, D_FEAT), lambda i: (i, 0)),
            pl.BlockSpec((D_FEAT, HIDDEN), lambda i: (0, 0)),
            pl.BlockSpec((1, HIDDEN), lambda i: (0, 0)),
        ],
        out_specs=pl.BlockSpec((_BN, HIDDEN), lambda i: (i, 0)),
        out_shape=jax.ShapeDtypeStruct((NPAD, HIDDEN), jnp.float32),
    )(x_pad, W1, b1.reshape(1, HIDDEN))


def _postA_body(pps_ref, pw_ref, pb_ref, fp_ref, cs_ref):
    i = pl.program_id(0)
    pps = pps_ref[...]
    pw = pw_ref[...]
    fp = jnp.tensordot(pps, pw[:, 0], axes=[[2], [0]])
    fp = fp.T + pb_ref[0, 0]
    rows = i * _BN + lax.broadcasted_iota(jnp.int32, (_BN, 1), 0)
    valid = rows < N_NODES
    part = jnp.sum(jnp.where(valid, fp, 0.0), axis=0, keepdims=True)

    @pl.when(i == 0)
    def _():
        cs_ref[...] = jnp.zeros_like(cs_ref)
    cs_ref[...] += part
    fp_ref[...] = fp


def _tc_postA(pps, proj_w, proj_b):
    nk = K_HOPS + 1
    return pl.pallas_call(
        _postA_body,
        grid=(NPAD // _BN,),
        in_specs=[
            pl.BlockSpec((nk, _BN, HIDDEN), lambda i: (0, i, 0)),
            pl.BlockSpec((HIDDEN, 1), lambda i: (0, 0)),
            pl.BlockSpec((1, 1), lambda i: (0, 0)),
        ],
        out_specs=[
            pl.BlockSpec((_BN, nk), lambda i: (i, 0)),
            pl.BlockSpec((1, nk), lambda i: (0, 0)),
        ],
        out_shape=[
            jax.ShapeDtypeStruct((NPAD, nk), jnp.float32),
            jax.ShapeDtypeStruct((1, nk), jnp.float32),
        ],
    )(pps, proj_w, proj_b.reshape(1, 1))


def _postB_body(pps_ref, fp_ref, cs_ref, prop_ref, w2_ref, b2_ref, out_ref):
    nk = K_HOPS + 1
    fp = fp_ref[...]
    m = cs_ref[...] / N_NODES
    sw = m + prop_ref[...]
    sw = sw / jnp.maximum(jnp.sqrt(jnp.sum(sw * sw)), 1e-12)
    sn = jnp.sqrt(jnp.sum(fp * fp, axis=1, keepdims=True))
    selfw = fp / jnp.maximum(sn, 1e-12)
    share = jnp.zeros((_BN, HIDDEN), jnp.float32)
    selfr = jnp.zeros((_BN, HIDDEN), jnp.float32)
    for k in range(nk):
        pk = pps_ref[k]
        share = share + sw[0, k] * pk
        selfr = selfr + selfw[:, k:k + 1] * pk
    out = share * 0.2 + selfr * 0.8
    t = jnp.tanh(out)
    logits = jnp.dot(t, w2_ref[...],
                     preferred_element_type=jnp.float32) + b2_ref[...]
    lmax = jnp.max(logits, axis=1, keepdims=True)
    ex = jnp.exp(logits - lmax)
    out_ref[...] = logits - lmax - jnp.log(jnp.sum(ex, axis=1, keepdims=True))


def _tc_postB(pps, fp, cs, prop_weight, W2, b2):
    nk = K_HOPS + 1
    return pl.pallas_call(
        _postB_body,
        grid=(NPAD // _BN,),
        in_specs=[
            pl.BlockSpec((nk, _BN, HIDDEN), lambda i: (0, i, 0)),
            pl.BlockSpec((_BN, nk), lambda i: (i, 0)),
            pl.BlockSpec((1, nk), lambda i: (0, 0)),
            pl.BlockSpec((1, nk), lambda i: (0, 0)),
            pl.BlockSpec((HIDDEN, NUM_CLASSES), lambda i: (0, 0)),
            pl.BlockSpec((1, NUM_CLASSES), lambda i: (0, 0)),
        ],
        out_specs=pl.BlockSpec((_BN, NUM_CLASSES), lambda i: (i, 0)),
        out_shape=jax.ShapeDtypeStruct((NPAD, NUM_CLASSES), jnp.float32),
    )(pps, fp, cs, prop_weight, W2, b2.reshape(1, NUM_CLASSES))


def kernel(x, edge_index, W1, b1, proj_w, proj_b, prop_weight, W2, b2):
    ei = edge_index.astype(jnp.int32)
    n_edges = ei.shape[1]
    n_real = n_edges + N_NODES
    loop = jnp.arange(N_NODES, dtype=jnp.int32)
    n_pad = EPAD - n_real
    row = jnp.concatenate([ei[0], loop, jnp.zeros((n_pad,), jnp.int32)])
    pad_cols = DUMMY + jnp.arange(n_pad, dtype=jnp.int32) % (NPAD - DUMMY)
    col = jnp.concatenate([ei[1], loop, pad_cols])
    row = row.reshape(16, NCHUNK, CH)
    col = col.reshape(16, NCHUNK, CH)

    x_pad = jnp.pad(x, ((0, NPAD - N_NODES), (0, 0)))
    h_pad = _tc_pre(x_pad, W1, b1)
    h_split = h_pad.reshape(NPAD, 2, HALF).transpose(1, 0, 2)

    u4 = _sc_prop(h_split, row, col)
    u_full = u4.transpose(0, 2, 1, 3).reshape(K_HOPS, NPAD, HIDDEN)
    pps = jnp.concatenate([h_pad[None], u_full], axis=0)

    fp, cs = _tc_postA(pps, proj_w, proj_b)
    out = _tc_postB(pps, fp, cs, prop_weight, W2, b2)
    return out[:N_NODES]


# speedup vs baseline: 18.8186x; 1.0114x over previous
import functools

import jax
import jax.numpy as jnp
from jax import lax
from jax.experimental import pallas as pl
from jax.experimental.pallas import tpu as pltpu
from jax.experimental.pallas import tpu_sc as plsc

N_NODES = 10000
D_FEAT = 128
HIDDEN = 64
K_HOPS = 8
NUM_CLASSES = 40

NPAD = 10240
TILE_NODES = NPAD // 16
HALF = HIDDEN // 2
CH = 512
NCHUNK = 41
EPT = CH * NCHUNK
EPAD = EPT * 16
DUMMY = N_NODES

_SCALE_CHUNKS = ((0, 160), (160, 160), (320, 160), (480, 160))


def _rsqrt(x):
    i = lax.bitcast_convert_type(x, jnp.int32)
    i = jnp.int32(0x5F3759DF) - (i >> 1)
    y = lax.bitcast_convert_type(i, jnp.float32)
    for _ in range(3):
        y = y * (1.5 - 0.5 * x * y * y)
    return y


def _sc_prop_body(h_hbm, row_hbm, col_hbm, ucur, u_out,
                  rowv, colv, gbuf, zbuf, s1, unext, gsem, ssem):
    c = lax.axis_index("c")
    sid = lax.axis_index("s")
    nb = sid * TILE_NODES

    def _wait_g(p):
        pltpu.make_async_copy(ucur.at[c, pl.ds(0, CH)], gbuf.at[p],
                              gsem.at[p]).wait()

    def _wait_s(p, sz=CH):
        pltpu.make_async_copy(gbuf.at[p, pl.ds(0, sz)],
                              unext.at[pl.ds(0, sz)], ssem.at[p]).wait()

    pltpu.sync_copy(row_hbm.at[sid], rowv)
    pltpu.sync_copy(col_hbm.at[sid], colv)

    def _zfill(i, _):
        zbuf[i, pl.ds(0, 16)] = jnp.zeros((16,), jnp.float32)
        zbuf[i, pl.ds(16, 16)] = jnp.zeros((16,), jnp.float32)
        return 0
    lax.fori_loop(0, 160, _zfill, 0)

    for q in range(4):
        pltpu.async_copy(zbuf, unext.at[pl.ds(nb + q * 160, 160)],
                         gsem.at[2])
    for q in range(4):
        pltpu.make_async_copy(zbuf, unext.at[pl.ds(nb, 160)],
                              gsem.at[2]).wait()
    plsc.subcore_barrier()

    def _ofill(i, _):
        gbuf[0, i, pl.ds(0, 16)] = jnp.full((16,), 1.0, jnp.float32)
        gbuf[0, i, pl.ds(16, 16)] = jnp.full((16,), 1.0, jnp.float32)
        return 0
    lax.fori_loop(0, CH, _ofill, 0)

    def _dscat(j, _):
        p = j % 2

        @pl.when(j >= 2)
        def _():
            _wait_s(p)
        pltpu.async_copy(gbuf.at[0], unext.at[colv.at[j]], ssem.at[p],
                         add=True)
        return 0
    lax.fori_loop(0, NCHUNK, _dscat, 0)
    _wait_s((NCHUNK - 2) % 2)
    _wait_s((NCHUNK - 1) % 2)
    plsc.subcore_barrier()

    pltpu.async_copy(unext.at[pl.ds(nb, 160)], gbuf.at[0, pl.ds(0, 160)],
                     gsem.at[0])
    for q, (base, sz) in enumerate(_SCALE_CHUNKS):
        p = q % 2
        pltpu.make_async_copy(unext.at[pl.ds(0, sz)],
                              gbuf.at[p, pl.ds(0, sz)], gsem.at[p]).wait()
        if q + 1 < len(_SCALE_CHUNKS):
            nbase, nsz = _SCALE_CHUNKS[q + 1]
            pltpu.async_copy(unext.at[pl.ds(nb + nbase, nsz)],
                             gbuf.at[1 - p, pl.ds(0, nsz)], gsem.at[1 - p])
        pltpu.async_copy(zbuf, unext.at[pl.ds(nb + base, sz)], gsem.at[2])

        def _sfill(r, _, base=base, p=p):
            dv = gbuf[p, r, pl.ds(0, 16)]
            s1[base + r] = _rsqrt(jnp.maximum(dv[0], 1.0))
            return 0
        lax.fori_loop(0, sz, _sfill, 0)
    for q in range(4):
        pltpu.make_async_copy(zbuf, unext.at[pl.ds(nb, 160)],
                              gsem.at[2]).wait()
    plsc.subcore_barrier()

    pltpu.async_copy(h_hbm.at[c, pl.ds(nb, 160)], gbuf.at[0, pl.ds(0, 160)],
                     gsem.at[0])
    for q, (base, sz) in enumerate(_SCALE_CHUNKS):
        p = q % 2
        pltpu.make_async_copy(h_hbm.at[c, pl.ds(0, sz)],
                              gbuf.at[p, pl.ds(0, sz)], gsem.at[p]).wait()
        if q >= 1:
            _wait_s(1 - p, 160)
        if q + 1 < len(_SCALE_CHUNKS):
            nbase, nsz = _SCALE_CHUNKS[q + 1]
            pltpu.async_copy(h_hbm.at[c, pl.ds(nb + nbase, nsz)],
                             gbuf.at[1 - p, pl.ds(0, nsz)], gsem.at[1 - p])

        def _ubody(r4, _, base=base, p=p):
            for d in range(4):
                r = r4 * 4 + d
                sv = jnp.broadcast_to(s1[base + r], (16,))
                for half in (0, 16):
                    gbuf[p, r, pl.ds(half, 16)] = (
                        gbuf[p, r, pl.ds(half, 16)] * sv)
            return 0
        lax.fori_loop(0, sz // 4, _ubody, 0)
        pltpu.async_copy(gbuf.at[p, pl.ds(0, sz)],
                         ucur.at[c, pl.ds(nb + base, sz)], ssem.at[p])
    _wait_s(1, 160)
    plsc.subcore_barrier()

    for k in range(K_HOPS):
        pltpu.async_copy(ucur.at[c].at[rowv.at[0]], gbuf.at[0], gsem.at[0])
        pltpu.async_copy(ucur.at[c].at[rowv.at[1]], gbuf.at[1], gsem.at[1])

        def _ebody(j, _):
            p = j % 3
            pn = (j + 2) % 3

            @pl.when(j + 2 < NCHUNK)
            def _():
                @pl.when(j >= 1)
                def _():
                    _wait_s(pn)
                pltpu.async_copy(ucur.at[c].at[rowv.at[j + 2]], gbuf.at[pn],
                                 gsem.at[pn])
            _wait_g(p)
            pltpu.async_copy(gbuf.at[p], unext.at[colv.at[j]], ssem.at[p],
                             add=True)
            return 0
        lax.fori_loop(0, NCHUNK, _ebody, 0)
        for jj in (NCHUNK - 3, NCHUNK - 2, NCHUNK - 1):
            _wait_s(jj % 3)
        plsc.subcore_barrier()

        pltpu.async_copy(unext.at[pl.ds(nb, 160)], gbuf.at[0, pl.ds(0, 160)],
                         gsem.at[0])
        for q, (base, sz) in enumerate(_SCALE_CHUNKS):
            p = q % 2
            pltpu.make_async_copy(unext.at[pl.ds(0, sz)],
                                  gbuf.at[p, pl.ds(0, sz)], gsem.at[p]).wait()
            if q >= 1:
                _wait_s(1 - p, 160)
                _wait_s(1 - p, 160)
            if q + 1 < len(_SCALE_CHUNKS):
                nbase, nsz = _SCALE_CHUNKS[q + 1]
                pltpu.async_copy(unext.at[pl.ds(nb + nbase, nsz)],
                                 gbuf.at[1 - p, pl.ds(0, nsz)],
                                 gsem.at[1 - p])
            pltpu.async_copy(zbuf, unext.at[pl.ds(nb + base, sz)],
                             gsem.at[2])

            def _sbody(r4, _, base=base, p=p):
                for d in range(4):
                    r = r4 * 4 + d
                    sv = jnp.broadcast_to(s1[base + r], (16,))
                    for half in (0, 16):
                        av = gbuf[p, r, pl.ds(half, 16)]
                        cv = av * sv
                        gbuf[p, 256 + r, pl.ds(half, 16)] = cv
                        gbuf[p, r, pl.ds(half, 16)] = cv * sv
                return 0
            lax.fori_loop(0, sz // 4, _sbody, 0)
            pltpu.async_copy(gbuf.at[p, pl.ds(0, sz)],
                             ucur.at[c, pl.ds(nb + base, sz)], ssem.at[p])
            pltpu.async_copy(gbuf.at[p, pl.ds(256, sz)],
                             u_out.at[k, c, pl.ds(nb + base, sz)], ssem.at[p])
        _wait_s(1, 160)
        _wait_s(1, 160)
        for q in range(4):
            pltpu.make_async_copy(zbuf, unext.at[pl.ds(nb, 160)],
                                  gsem.at[2]).wait()
        plsc.subcore_barrier()


def _sc_prop(h_split, row, col):
    mesh = plsc.VectorSubcoreMesh(core_axis_name="c", subcore_axis_name="s")
    fn = functools.partial(
        pl.kernel,
        mesh=mesh,
        compiler_params=pltpu.CompilerParams(use_tc_tiling_on_sc=False),
        out_type=jax.ShapeDtypeStruct((K_HOPS, 2, NPAD, HALF), jnp.float32),
        scratch_types=[
            pltpu.VMEM((NCHUNK, CH), jnp.int32),
            pltpu.VMEM((NCHUNK, CH), jnp.int32),
            pltpu.VMEM((3, CH, HALF), jnp.float32),
            pltpu.VMEM((160, HALF), jnp.float32),
            pltpu.SMEM((TILE_NODES,), jnp.float32),
            pltpu.VMEM_SHARED((NPAD, HALF), jnp.float32),
            pltpu.SemaphoreType.DMA((3,)),
            pltpu.SemaphoreType.DMA((3,)),
        ],
    )(_sc_prop_body)
    ucur_ref = jax.new_ref(jnp.zeros((2, NPAD, HALF), jnp.float32))
    return fn(h_split, row, col, ucur_ref)


_BN = 1280


def _pre_body(x_ref, w_ref, b_ref, h_ref):
    h = jnp.dot(x_ref[...], w_ref[...],
                preferred_element_type=jnp.float32) + b_ref[...]
    alpha = 1.6732632423543772848170429916717
    scale = 1.0507009873554804934193349852946
    neg = alpha * (jnp.exp(jnp.minimum(h, 0.0)) - 1.0)
    h_ref[...] = scale * jnp.where(h > 0.0, h, neg)


def _tc_pre(x_pad, W1, b1):
    return pl.pallas_call(
        _pre_body,
        grid=(NPAD // _BN,),
        in_specs=[
            pl.BlockSpec((_BN, D_FEAT), lambda i: (i, 0)),
            pl.BlockSpec((D_FEAT, HIDDEN), lambda i: (0, 0)),
            pl.BlockSpec((1, HIDDEN), lambda i: (0, 0)),
        ],
        out_specs=pl.BlockSpec((_BN, HIDDEN), lambda i: (i, 0)),
        out_shape=jax.ShapeDtypeStruct((NPAD, HIDDEN), jnp.float32),
    )(x_pad, W1, b1.reshape(1, HIDDEN))


def _postA_body(pps_ref, pw_ref, pb_ref, fp_ref, cs_ref):
    i = pl.program_id(0)
    pps = pps_ref[...]
    pw = pw_ref[...]
    fp = jnp.tensordot(pps, pw[:, 0], axes=[[2], [0]])
    fp = fp.T + pb_ref[0, 0]
    rows = i * _BN + lax.broadcasted_iota(jnp.int32, (_BN, 1), 0)
    valid = rows < N_NODES
    part = jnp.sum(jnp.where(valid, fp, 0.0), axis=0, keepdims=True)

    @pl.when(i == 0)
    def _():
        cs_ref[...] = jnp.zeros_like(cs_ref)
    cs_ref[...] += part
    fp_ref[...] = fp


def _tc_postA(pps, proj_w, proj_b):
    nk = K_HOPS + 1
    return pl.pallas_call(
        _postA_body,
        grid=(NPAD // _BN,),
        in_specs=[
            pl.BlockSpec((nk, _BN, HIDDEN), lambda i: (0, i, 0)),
            pl.BlockSpec((HIDDEN, 1), lambda i: (0, 0)),
            pl.BlockSpec((1, 1), lambda i: (0, 0)),
        ],
        out_specs=[
            pl.BlockSpec((_BN, nk), lambda i: (i, 0)),
            pl.BlockSpec((1, nk), lambda i: (0, 0)),
        ],
        out_shape=[
            jax.ShapeDtypeStruct((NPAD, nk), jnp.float32),
            jax.ShapeDtypeStruct((1, nk), jnp.float32),
        ],
    )(pps, proj_w, proj_b.reshape(1, 1))


def _postB_body(pps_ref, fp_ref, cs_ref, prop_ref, w2_ref, b2_ref, out_ref):
    nk = K_HOPS + 1
    fp = fp_ref[...]
    m = cs_ref[...] / N_NODES
    sw = m + prop_ref[...]
    sw = sw / jnp.maximum(jnp.sqrt(jnp.sum(sw * sw)), 1e-12)
    sn = jnp.sqrt(jnp.sum(fp * fp, axis=1, keepdims=True))
    selfw = fp / jnp.maximum(sn, 1e-12)
    share = jnp.zeros((_BN, HIDDEN), jnp.float32)
    selfr = jnp.zeros((_BN, HIDDEN), jnp.float32)
    for k in range(nk):
        pk = pps_ref[k]
        share = share + sw[0, k] * pk
        selfr = selfr + selfw[:, k:k + 1] * pk
    out = share * 0.2 + selfr * 0.8
    t = jnp.tanh(out)
    logits = jnp.dot(t, w2_ref[...],
                     preferred_element_type=jnp.float32) + b2_ref[...]
    lmax = jnp.max(logits, axis=1, keepdims=True)
    ex = jnp.exp(logits - lmax)
    out_ref[...] = logits - lmax - jnp.log(jnp.sum(ex, axis=1, keepdims=True))


def _tc_postB(pps, fp, cs, prop_weight, W2, b2):
    nk = K_HOPS + 1
    return pl.pallas_call(
        _postB_body,
        grid=(NPAD // _BN,),
        in_specs=[
            pl.BlockSpec((nk, _BN, HIDDEN), lambda i: (0, i, 0)),
            pl.BlockSpec((_BN, nk), lambda i: (i, 0)),
            pl.BlockSpec((1, nk), lambda i: (0, 0)),
            pl.BlockSpec((1, nk), lambda i: (0, 0)),
            pl.BlockSpec((HIDDEN, NUM_CLASSES), lambda i: (0, 0)),
            pl.BlockSpec((1, NUM_CLASSES), lambda i: (0, 0)),
        ],
        out_specs=pl.BlockSpec((_BN, NUM_CLASSES), lambda i: (i, 0)),
        out_shape=jax.ShapeDtypeStruct((NPAD, NUM_CLASSES), jnp.float32),
    )(pps, fp, cs, prop_weight, W2, b2.reshape(1, NUM_CLASSES))


def kernel(x, edge_index, W1, b1, proj_w, proj_b, prop_weight, W2, b2):
    ei = edge_index.astype(jnp.int32)
    n_edges = ei.shape[1]
    n_real = n_edges + N_NODES
    loop = jnp.arange(N_NODES, dtype=jnp.int32)
    n_pad = EPAD - n_real
    row = jnp.concatenate([ei[0], loop, jnp.zeros((n_pad,), jnp.int32)])
    pad_cols = DUMMY + jnp.arange(n_pad, dtype=jnp.int32) % (NPAD - DUMMY)
    col = jnp.concatenate([ei[1], loop, pad_cols])
    row = row.reshape(16, NCHUNK, CH)
    col = col.reshape(16, NCHUNK, CH)

    x_pad = jnp.pad(x, ((0, NPAD - N_NODES), (0, 0)))
    h_pad = _tc_pre(x_pad, W1, b1)
    h_split = h_pad.reshape(NPAD, 2, HALF).transpose(1, 0, 2)

    u4 = _sc_prop(h_split, row, col)
    u_full = u4.transpose(0, 2, 1, 3).reshape(K_HOPS, NPAD, HIDDEN)
    pps = jnp.concatenate([h_pad[None], u_full], axis=0)

    fp, cs = _tc_postA(pps, proj_w, proj_b)
    out = _tc_postB(pps, fp, cs, prop_weight, W2, b2)
    return out[:N_NODES]
